# Initial kernel scaffold; baseline (speedup 1.0000x reference)
import jax
import jax.numpy as jnp
from jax.experimental import pallas as pl


def kernel(in_feat, edge_index, W1, b1, W2, b2, W3, b3, W4, b4):
    raise NotImplementedError("write your pallas kernel here")


# speedup vs baseline: 10.6502x; 10.6502x over previous
import functools

import jax
import jax.numpy as jnp
from jax import lax
from jax.experimental import pallas as pl
from jax.experimental.pallas import tpu as pltpu
from jax.experimental.pallas import tpu_sc as plsc

_NC = 2
_NS = 16


def _mesh():
    return plsc.VectorSubcoreMesh(core_axis_name="c", subcore_axis_name="s")


def _make_deg_kernel(n_pad, ch):
    rpt = n_pad // _NS

    @functools.partial(
        pl.kernel,
        mesh=_mesh(),
        out_type=jax.ShapeDtypeStruct((_NC, n_pad), jnp.float32),
        scratch_types=[
            pltpu.VMEM((ch, 128), jnp.int32),
            pltpu.VMEM((128,), jnp.float32),
            pltpu.VMEM_SHARED((n_pad,), jnp.float32),
        ],
    )
    def deg_kernel(zeros_hbm, row_hbm, out_hbm, row_v, ones_v, deg_sh):
        cid = lax.axis_index("c")
        sid = lax.axis_index("s")
        wid = sid * _NC + cid
        pltpu.sync_copy(zeros_hbm.at[pl.ds(sid * rpt, rpt)],
                        deg_sh.at[pl.ds(sid * rpt, rpt)])
        pltpu.sync_copy(row_hbm.at[wid], row_v)
        for i in range(8):
            ones_v[pl.ds(i * 16, 16)] = jnp.ones((16,), jnp.float32)
        plsc.subcore_barrier()

        def body(j, carry):
            pltpu.sync_copy(ones_v, deg_sh.at[row_v.at[j]], add=True)
            return carry

        lax.fori_loop(0, ch, body, 0)
        plsc.subcore_barrier()
        pltpu.sync_copy(deg_sh.at[pl.ds(sid * rpt, rpt)],
                        out_hbm.at[cid, pl.ds(sid * rpt, rpt)])

    return deg_kernel


def _make_spmm_kernel(n_pad, ch, f):
    rpt = n_pad // _NS

    @functools.partial(
        pl.kernel,
        mesh=_mesh(),
        out_type=jax.ShapeDtypeStruct((_NC, n_pad, f), jnp.float32),
        scratch_types=[
            pltpu.VMEM((ch, 128), jnp.int32),
            pltpu.VMEM((ch, 128), jnp.int32),
            pltpu.VMEM((128, f), jnp.float32),
            pltpu.VMEM_SHARED((n_pad, f), jnp.float32),
            pltpu.SemaphoreType.DMA,
        ],
    )
    def spmm_kernel(g_hbm, zeros_hbm, col_hbm, row_hbm, out_hbm,
                    col_v, row_v, rows_v, acc_sh, sem):
        cid = lax.axis_index("c")
        sid = lax.axis_index("s")
        wid = sid * _NC + cid
        pltpu.sync_copy(zeros_hbm.at[pl.ds(sid * rpt, rpt)],
                        acc_sh.at[pl.ds(sid * rpt, rpt)])
        pltpu.sync_copy(col_hbm.at[wid], col_v)
        pltpu.sync_copy(row_hbm.at[wid], row_v)
        plsc.subcore_barrier()

        def body(j, carry):
            pltpu.async_copy(g_hbm.at[col_v.at[j]], rows_v, sem).wait()
            pltpu.sync_copy(rows_v, acc_sh.at[row_v.at[j]], add=True)
            return carry

        lax.fori_loop(0, ch, body, 0)
        plsc.subcore_barrier()
        pltpu.sync_copy(acc_sh.at[pl.ds(sid * rpt, rpt)],
                        out_hbm.at[cid, pl.ds(sid * rpt, rpt)])

    return spmm_kernel


def _dg(a, b):
    return lax.dot_general(a, b, (((1,), (1,)), ((), ())),
                           preferred_element_type=jnp.float32)


def _mlp_body(x_ref, w1_ref, b1_ref, w2_ref, b2_ref, h_ref):
    t = jnp.maximum(_dg(x_ref[...], w1_ref[...]) + b1_ref[...], 0.0)
    h_ref[...] = jnp.maximum(_dg(t, w2_ref[...]) + b2_ref[...], 0.0)


def _scale_body(degp_ref, h_ref, dis_ref, g1_ref):
    deg = degp_ref[0] + degp_ref[1]
    safe = jnp.where(deg > 0.0, deg, 1.0)
    dis = jnp.where(deg > 0.0, lax.rsqrt(safe), 0.0)
    dis_ref[...] = dis
    g1_ref[...] = dis * h_ref[...]


def _combine_body(sp_ref, h_ref, dis_ref, p_ref, g2_ref):
    s = sp_ref[0] + sp_ref[1]
    dis = dis_ref[...]
    p = h_ref[...] - dis * s
    p_ref[...] = p
    g2_ref[...] = dis * p


def _final_body(sp_ref, h_ref, p_ref, dis_ref, w3_ref, b3_ref,
                w4_ref, b4_ref, o_ref):
    f = h_ref.shape[1]
    q = p_ref[...] - dis_ref[...] * (sp_ref[0] + sp_ref[1])
    h = h_ref[...]
    p = p_ref[...]
    o0 = 3.0 * h - 3.0 * p + 0.75 * q
    o1 = 3.0 * p - 1.5 * q
    o2 = 0.75 * q
    w3 = w3_ref[...]
    z = (_dg(o0, w3[:, 0:f]) + _dg(o1, w3[:, f:2 * f])
         + _dg(o2, w3[:, 2 * f:3 * f]) + b3_ref[...])
    z = jnp.maximum(z, 0.0)
    o_ref[...] = _dg(z, w4_ref[...]) + b4_ref[...]


def _row_blocked_call(body, n_pad, bm, in_arrays, in_specs, n_out, f):
    out_specs = [pl.BlockSpec((bm, f), lambda i: (i, 0))
                 for _ in range(n_out)]
    out_shape = [jax.ShapeDtypeStruct((n_pad, f), jnp.float32)
                 for _ in range(n_out)]
    if n_out == 1:
        out_specs, out_shape = out_specs[0], out_shape[0]
    return pl.pallas_call(
        body,
        grid=(n_pad // bm,),
        in_specs=in_specs,
        out_specs=out_specs,
        out_shape=out_shape,
    )(*in_arrays)


def kernel(in_feat, edge_index, W1, b1, W2, b2, W3, b3, W4, b4):
    n, f = in_feat.shape
    e = edge_index.shape[1]
    nw = _NC * _NS
    bm = 1024
    n_pad = -(-(n + 1) // bm) * bm
    ch = -(-e // (nw * 128))
    e_pad = nw * ch * 128

    ei = edge_index.astype(jnp.int32)
    pad = e_pad - e
    row = jnp.concatenate([ei[0], jnp.full((pad,), n, jnp.int32)])
    col = jnp.concatenate([ei[1], jnp.full((pad,), n, jnp.int32)])
    row3 = row.reshape(nw, ch, 128)
    col3 = col.reshape(nw, ch, 128)

    x_pad = jnp.pad(in_feat, ((0, n_pad - n), (0, 0)))
    zeros2 = jnp.zeros((n_pad, f), jnp.float32)
    zeros1 = jnp.zeros((n_pad,), jnp.float32)
    b1r = b1.reshape(1, f)
    b2r = b2.reshape(1, f)
    b3r = b3.reshape(1, f)
    w4p = jnp.pad(W4, ((0, f - W4.shape[0]), (0, 0)))
    b4p = jnp.pad(b4, (0, f - b4.shape[0])).reshape(1, f)

    deg_k = _make_deg_kernel(n_pad, ch)
    spmm_k = _make_spmm_kernel(n_pad, ch, f)

    degp = deg_k(zeros1, row3)

    h = _row_blocked_call(
        _mlp_body, n_pad, bm,
        (x_pad, W1, b1r, W2, b2r),
        [pl.BlockSpec((bm, f), lambda i: (i, 0)),
         pl.BlockSpec((f, f), lambda i: (0, 0)),
         pl.BlockSpec((1, f), lambda i: (0, 0)),
         pl.BlockSpec((f, f), lambda i: (0, 0)),
         pl.BlockSpec((1, f), lambda i: (0, 0))],
        1, f)

    dis, g1 = pl.pallas_call(
        _scale_body,
        grid=(n_pad // bm,),
        in_specs=[pl.BlockSpec((_NC, bm, 1), lambda i: (0, i, 0)),
                  pl.BlockSpec((bm, f), lambda i: (i, 0))],
        out_specs=[pl.BlockSpec((bm, 1), lambda i: (i, 0)),
                   pl.BlockSpec((bm, f), lambda i: (i, 0))],
        out_shape=[jax.ShapeDtypeStruct((n_pad, 1), jnp.float32),
                   jax.ShapeDtypeStruct((n_pad, f), jnp.float32)],
    )(degp.reshape(_NC, n_pad, 1), h)

    s1 = spmm_k(g1, zeros2, col3, row3)

    p, g2 = pl.pallas_call(
        _combine_body,
        grid=(n_pad // bm,),
        in_specs=[pl.BlockSpec((_NC, bm, f), lambda i: (0, i, 0)),
                  pl.BlockSpec((bm, f), lambda i: (i, 0)),
                  pl.BlockSpec((bm, 1), lambda i: (i, 0))],
        out_specs=[pl.BlockSpec((bm, f), lambda i: (i, 0)),
                   pl.BlockSpec((bm, f), lambda i: (i, 0))],
        out_shape=[jax.ShapeDtypeStruct((n_pad, f), jnp.float32),
                   jax.ShapeDtypeStruct((n_pad, f), jnp.float32)],
    )(s1, h, dis)

    s2 = spmm_k(g2, zeros2, col3, row3)

    out = pl.pallas_call(
        _final_body,
        grid=(n_pad // bm,),
        in_specs=[pl.BlockSpec((_NC, bm, f), lambda i: (0, i, 0)),
                  pl.BlockSpec((bm, f), lambda i: (i, 0)),
                  pl.BlockSpec((bm, f), lambda i: (i, 0)),
                  pl.BlockSpec((bm, 1), lambda i: (i, 0)),
                  pl.BlockSpec((f, 3 * f), lambda i: (0, 0)),
                  pl.BlockSpec((1, f), lambda i: (0, 0)),
                  pl.BlockSpec((f, f), lambda i: (0, 0)),
                  pl.BlockSpec((1, f), lambda i: (0, 0))],
        out_specs=pl.BlockSpec((bm, f), lambda i: (i, 0)),
        out_shape=jax.ShapeDtypeStruct((n_pad, f), jnp.float32),
    )(s2, h, p, dis, W3, b3r, w4p, b4p)

    return out[:n, :W4.shape[0]]


# speedup vs baseline: 12.1379x; 1.1397x over previous
import functools

import jax
import jax.numpy as jnp
from jax import lax
from jax.experimental import pallas as pl
from jax.experimental.pallas import tpu as pltpu
from jax.experimental.pallas import tpu_sc as plsc

_NC = 2
_NS = 16


def _mesh():
    return plsc.VectorSubcoreMesh(core_axis_name="c", subcore_axis_name="s")


def _make_deg_kernel(n_pad, ch):
    rpt = n_pad // _NS

    @functools.partial(
        pl.kernel,
        mesh=_mesh(),
        out_type=jax.ShapeDtypeStruct((_NC, n_pad), jnp.float32),
        scratch_types=[
            pltpu.VMEM((ch, 128), jnp.int32),
            pltpu.VMEM((128,), jnp.float32),
            pltpu.VMEM_SHARED((n_pad,), jnp.float32),
        ],
    )
    def deg_kernel(zeros_hbm, row_hbm, out_hbm, row_v, ones_v, deg_sh):
        cid = lax.axis_index("c")
        sid = lax.axis_index("s")
        wid = sid * _NC + cid
        pltpu.sync_copy(zeros_hbm.at[pl.ds(sid * rpt, rpt)],
                        deg_sh.at[pl.ds(sid * rpt, rpt)])
        pltpu.sync_copy(row_hbm.at[wid], row_v)
        for i in range(8):
            ones_v[pl.ds(i * 16, 16)] = jnp.ones((16,), jnp.float32)
        plsc.subcore_barrier()

        def body(j, carry):
            pltpu.sync_copy(ones_v, deg_sh.at[row_v.at[j]], add=True)
            return carry

        lax.fori_loop(0, ch, body, 0)
        plsc.subcore_barrier()
        pltpu.sync_copy(deg_sh.at[pl.ds(sid * rpt, rpt)],
                        out_hbm.at[cid, pl.ds(sid * rpt, rpt)])

    return deg_kernel


def _make_spmm_kernel(n_pad, ch, f):
    rpt = n_pad // _NS

    @functools.partial(
        pl.kernel,
        mesh=_mesh(),
        out_type=jax.ShapeDtypeStruct((_NC, n_pad, f), jnp.float32),
        scratch_types=[
            pltpu.VMEM((ch, 128), jnp.int32),
            pltpu.VMEM((ch, 128), jnp.int32),
            pltpu.VMEM((64, f), jnp.float32),
            pltpu.VMEM((64, f), jnp.float32),
            pltpu.VMEM_SHARED((n_pad, f), jnp.float32),
            pltpu.SemaphoreType.DMA,
            pltpu.SemaphoreType.DMA,
        ],
    )
    def spmm_kernel(g_hbm, zeros_hbm, col_hbm, row_hbm, out_hbm,
                    col_v, row_v, buf0, buf1, acc_sh, sem0, sem1):
        bufs = (buf0, buf1)
        sems = (sem0, sem1)
        cid = lax.axis_index("c")
        sid = lax.axis_index("s")
        wid = sid * _NC + cid
        pltpu.sync_copy(zeros_hbm.at[pl.ds(sid * rpt, rpt)],
                        acc_sh.at[pl.ds(sid * rpt, rpt)])
        pltpu.sync_copy(col_hbm.at[wid], col_v)
        pltpu.sync_copy(row_hbm.at[wid], row_v)
        plsc.subcore_barrier()

        for h in range(2):
            pltpu.async_copy(g_hbm.at[col_v.at[0, pl.ds(h * 64, 64)]],
                             bufs[h], sems[h])

        def body(r, carry):
            rn = jnp.minimum(r + 1, ch - 1)
            for h in range(2):
                pltpu.make_async_copy(
                    g_hbm.at[col_v.at[r, pl.ds(h * 64, 64)]],
                    bufs[h], sems[h]).wait()
                pltpu.sync_copy(bufs[h],
                                acc_sh.at[row_v.at[r, pl.ds(h * 64, 64)]],
                                add=True)
                pltpu.async_copy(g_hbm.at[col_v.at[rn, pl.ds(h * 64, 64)]],
                                 bufs[h], sems[h])
            return carry

        lax.fori_loop(0, ch, body, 0)
        for h in range(2):
            pltpu.make_async_copy(
                g_hbm.at[col_v.at[0, pl.ds(h * 64, 64)]],
                bufs[h], sems[h]).wait()
        plsc.subcore_barrier()
        pltpu.sync_copy(acc_sh.at[pl.ds(sid * rpt, rpt)],
                        out_hbm.at[cid, pl.ds(sid * rpt, rpt)])

    return spmm_kernel


def _dg(a, b):
    return lax.dot_general(a, b, (((1,), (1,)), ((), ())),
                           preferred_element_type=jnp.float32)


def _mlp_body(x_ref, w1_ref, b1_ref, w2_ref, b2_ref, h_ref):
    t = jnp.maximum(_dg(x_ref[...], w1_ref[...]) + b1_ref[...], 0.0)
    h_ref[...] = jnp.maximum(_dg(t, w2_ref[...]) + b2_ref[...], 0.0)


def _scale_body(degp_ref, h_ref, dis_ref, g1_ref):
    deg = degp_ref[0] + degp_ref[1]
    safe = jnp.where(deg > 0.0, deg, 1.0)
    dis = jnp.where(deg > 0.0, lax.rsqrt(safe), 0.0)
    dis_ref[...] = dis
    g1_ref[...] = dis * h_ref[...]


def _combine_body(sp_ref, h_ref, dis_ref, p_ref, g2_ref):
    s = sp_ref[0] + sp_ref[1]
    dis = dis_ref[...]
    p = h_ref[...] - dis * s
    p_ref[...] = p
    g2_ref[...] = dis * p


def _final_body(sp_ref, h_ref, p_ref, dis_ref, w3_ref, b3_ref,
                w4_ref, b4_ref, o_ref):
    f = h_ref.shape[1]
    q = p_ref[...] - dis_ref[...] * (sp_ref[0] + sp_ref[1])
    h = h_ref[...]
    p = p_ref[...]
    o0 = 3.0 * h - 3.0 * p + 0.75 * q
    o1 = 3.0 * p - 1.5 * q
    o2 = 0.75 * q
    w3 = w3_ref[...]
    z = (_dg(o0, w3[:, 0:f]) + _dg(o1, w3[:, f:2 * f])
         + _dg(o2, w3[:, 2 * f:3 * f]) + b3_ref[...])
    z = jnp.maximum(z, 0.0)
    o_ref[...] = _dg(z, w4_ref[...]) + b4_ref[...]


def _row_blocked_call(body, n_pad, bm, in_arrays, in_specs, n_out, f):
    out_specs = [pl.BlockSpec((bm, f), lambda i: (i, 0))
                 for _ in range(n_out)]
    out_shape = [jax.ShapeDtypeStruct((n_pad, f), jnp.float32)
                 for _ in range(n_out)]
    if n_out == 1:
        out_specs, out_shape = out_specs[0], out_shape[0]
    return pl.pallas_call(
        body,
        grid=(n_pad // bm,),
        in_specs=in_specs,
        out_specs=out_specs,
        out_shape=out_shape,
    )(*in_arrays)


def kernel(in_feat, edge_index, W1, b1, W2, b2, W3, b3, W4, b4):
    n, f = in_feat.shape
    e = edge_index.shape[1]
    nw = _NC * _NS
    bm = 1024
    n_pad = -(-(n + 1) // bm) * bm
    ch = -(-e // (nw * 128))
    e_pad = nw * ch * 128

    ei = edge_index.astype(jnp.int32)
    pad = e_pad - e
    row = jnp.concatenate([ei[0], jnp.full((pad,), n, jnp.int32)])
    col = jnp.concatenate([ei[1], jnp.full((pad,), n, jnp.int32)])
    row3 = row.reshape(nw, ch, 128)
    col3 = col.reshape(nw, ch, 128)

    x_pad = jnp.pad(in_feat, ((0, n_pad - n), (0, 0)))
    zeros2 = jnp.zeros((n_pad, f), jnp.float32)
    zeros1 = jnp.zeros((n_pad,), jnp.float32)
    b1r = b1.reshape(1, f)
    b2r = b2.reshape(1, f)
    b3r = b3.reshape(1, f)
    w4p = jnp.pad(W4, ((0, f - W4.shape[0]), (0, 0)))
    b4p = jnp.pad(b4, (0, f - b4.shape[0])).reshape(1, f)

    deg_k = _make_deg_kernel(n_pad, ch)
    spmm_k = _make_spmm_kernel(n_pad, ch, f)

    degp = deg_k(zeros1, row3)

    h = _row_blocked_call(
        _mlp_body, n_pad, bm,
        (x_pad, W1, b1r, W2, b2r),
        [pl.BlockSpec((bm, f), lambda i: (i, 0)),
         pl.BlockSpec((f, f), lambda i: (0, 0)),
         pl.BlockSpec((1, f), lambda i: (0, 0)),
         pl.BlockSpec((f, f), lambda i: (0, 0)),
         pl.BlockSpec((1, f), lambda i: (0, 0))],
        1, f)

    dis, g1 = pl.pallas_call(
        _scale_body,
        grid=(n_pad // bm,),
        in_specs=[pl.BlockSpec((_NC, bm, 1), lambda i: (0, i, 0)),
                  pl.BlockSpec((bm, f), lambda i: (i, 0))],
        out_specs=[pl.BlockSpec((bm, 1), lambda i: (i, 0)),
                   pl.BlockSpec((bm, f), lambda i: (i, 0))],
        out_shape=[jax.ShapeDtypeStruct((n_pad, 1), jnp.float32),
                   jax.ShapeDtypeStruct((n_pad, f), jnp.float32)],
    )(degp.reshape(_NC, n_pad, 1), h)

    s1 = spmm_k(g1, zeros2, col3, row3)

    p, g2 = pl.pallas_call(
        _combine_body,
        grid=(n_pad // bm,),
        in_specs=[pl.BlockSpec((_NC, bm, f), lambda i: (0, i, 0)),
                  pl.BlockSpec((bm, f), lambda i: (i, 0)),
                  pl.BlockSpec((bm, 1), lambda i: (i, 0))],
        out_specs=[pl.BlockSpec((bm, f), lambda i: (i, 0)),
                   pl.BlockSpec((bm, f), lambda i: (i, 0))],
        out_shape=[jax.ShapeDtypeStruct((n_pad, f), jnp.float32),
                   jax.ShapeDtypeStruct((n_pad, f), jnp.float32)],
    )(s1, h, dis)

    s2 = spmm_k(g2, zeros2, col3, row3)

    out = pl.pallas_call(
        _final_body,
        grid=(n_pad // bm,),
        in_specs=[pl.BlockSpec((_NC, bm, f), lambda i: (0, i, 0)),
                  pl.BlockSpec((bm, f), lambda i: (i, 0)),
                  pl.BlockSpec((bm, f), lambda i: (i, 0)),
                  pl.BlockSpec((bm, 1), lambda i: (i, 0)),
                  pl.BlockSpec((f, 3 * f), lambda i: (0, 0)),
                  pl.BlockSpec((1, f), lambda i: (0, 0)),
                  pl.BlockSpec((f, f), lambda i: (0, 0)),
                  pl.BlockSpec((1, f), lambda i: (0, 0))],
        out_specs=pl.BlockSpec((bm, f), lambda i: (i, 0)),
        out_shape=jax.ShapeDtypeStruct((n_pad, f), jnp.float32),
    )(s2, h, p, dis, W3, b3r, w4p, b4p)

    return out[:n, :W4.shape[0]]
